# baseline (speedup 1.0000x reference)
import jax
import jax.numpy as jnp
from jax.experimental import pallas as pl
from jax.experimental.pallas import tpu as pltpu

_B = 64
_D = 256
_L = 2000
_M = 16
_N = _L * _M
_T = 0.07
_HARD_K = 50
_LAMDA = 0.5
_BLK = 8000
_NB = _N // (2 * _BLK)


def _loss_kernel(feats_ref, lab_ref, cam_ref, cena_ref, cenb_ref, out_ref,
                 x_ref, camm_ref, jdiv_ref, acc_ref, hard_ref):
    i = pl.program_id(0)
    lab = lab_ref[...]
    cam = cam_ref[...]

    @pl.when(i == 0)
    def _init():
        f = feats_ref[...]
        x_ref[...] = f / jnp.sqrt(jnp.sum(f * f, axis=1, keepdims=True))
        j = jax.lax.broadcasted_iota(jnp.int32, (_B, _BLK), 1)
        camm_ref[...] = (jnp.bitwise_and(j, _M - 1) == cam).astype(jnp.float32)
        jdiv_ref[...] = jax.lax.shift_right_logical(j, 4)
        acc_ref[...] = jnp.zeros_like(acc_ref)
        jh = jax.lax.broadcasted_iota(jnp.int32, (_B, 128), 1)
        lab16 = lab * _M
        hmask = (((jh < lab16) & (jh < _HARD_K)) |
                 ((jh >= lab16 + _M) & (jh < _HARD_K + _M)))
        hard_ref[...] = jnp.where(hmask, 1.0, 0.0)

    x = x_ref[...]
    camm = camm_ref[...]
    jdiv = jdiv_ref[...]
    zero = jnp.zeros((_B, _BLK), jnp.float32)
    fd = pd = up = jnp.zeros((_B, 1), jnp.float32)
    for half, c_ref in ((0, cena_ref), (1, cenb_ref)):
        s = jax.lax.dot_general(x, c_ref[...], (((1,), (1,)), ((), ())),
                                preferred_element_type=jnp.float32)
        e = jnp.exp(s * (1.0 / _T))
        ecam = e * camm
        blk0 = i + half * _NB
        pos_mask = jdiv == (lab - blk0 * (_BLK // _M))
        fd = fd + jnp.sum(ecam, axis=1, keepdims=True)
        pd = pd + jnp.sum(jnp.where(pos_mask, e, zero), axis=1, keepdims=True)
        up = up + jnp.sum(jnp.where(pos_mask, ecam, zero), axis=1,
                          keepdims=True)
        if half == 0:
            hscale = jnp.where(i == 0, 1.0, 0.0)
            hd = jnp.sum(e[:, :128] * hard_ref[...], axis=1,
                         keepdims=True) * hscale
    acc_ref[...] += jnp.concatenate([fd, pd, up, hd], axis=1)

    @pl.when(i == _NB - 1)
    def _finish():
        acc = acc_ref[...]
        log_up = jnp.log(acc[:, 2:3])
        log_fd = jnp.log(acc[:, 0:1])
        log_pd = jnp.log(acc[:, 1:2] + acc[:, 3:4])
        intra = -jnp.sum(log_up - log_fd)
        inter = -jnp.sum(log_up - log_pd)
        out_ref[...] = jnp.concatenate(
            [intra.reshape(1, 1), (_LAMDA * inter).reshape(1, 1)], axis=1)


def kernel(feats, centers, labels, camids, epoch):
    lab = labels.reshape(_B, 1).astype(jnp.int32)
    cam = camids.reshape(_B, 1).astype(jnp.int32)
    out = pl.pallas_call(
        _loss_kernel,
        grid=(_NB,),
        in_specs=[
            pl.BlockSpec((_B, _D), lambda i: (0, 0)),
            pl.BlockSpec((_B, 1), lambda i: (0, 0)),
            pl.BlockSpec((_B, 1), lambda i: (0, 0)),
            pl.BlockSpec((_BLK, _D), lambda i: (i, 0)),
            pl.BlockSpec((_BLK, _D), lambda i: (i + _NB, 0)),
        ],
        out_specs=pl.BlockSpec((1, 2), lambda i: (0, 0)),
        out_shape=jax.ShapeDtypeStruct((1, 2), jnp.float32),
        scratch_shapes=[
            pltpu.VMEM((_B, _D), jnp.float32),
            pltpu.VMEM((_B, _BLK), jnp.float32),
            pltpu.VMEM((_B, _BLK), jnp.int32),
            pltpu.VMEM((_B, 4), jnp.float32),
            pltpu.VMEM((_B, 128), jnp.float32),
        ],
        compiler_params=pltpu.CompilerParams(
            dimension_semantics=("arbitrary",)),
    )(feats, lab, cam, centers, centers)
    gate = (jnp.asarray(epoch) >= 5).astype(jnp.float32)
    return out.reshape(2) * gate


# speedup vs baseline: 1.1207x; 1.1207x over previous
import jax
import jax.numpy as jnp
from jax.experimental import pallas as pl
from jax.experimental.pallas import tpu as pltpu

_B = 64
_D = 256
_L = 2000
_M = 16
_N = _L * _M
_T = 0.07
_HARD_K = 50
_LAMDA = 0.5
_BLK = 16000
_NB = _N // _BLK


def _loss_kernel(feats_ref, lab_ref, cam_ref, gate_ref, cen_ref, out_ref,
                 x_ref, camm_ref, jdiv_ref, acc_ref, hard_ref):
    i = pl.program_id(0)
    lab = lab_ref[...]
    cam = cam_ref[...]

    @pl.when(i == 0)
    def _init():
        f = feats_ref[...]
        x_ref[...] = f / jnp.sqrt(jnp.sum(f * f, axis=1, keepdims=True))
        j = jax.lax.broadcasted_iota(jnp.int32, (_B, _BLK), 1)
        camm_ref[...] = (jnp.bitwise_and(j, _M - 1) == cam).astype(jnp.float32)
        jdiv_ref[...] = jax.lax.shift_right_logical(j, 4)
        acc_ref[...] = jnp.zeros_like(acc_ref)
        jh = jax.lax.broadcasted_iota(jnp.int32, (_B, 128), 1)
        lab16 = lab * _M
        hmask = (((jh < lab16) & (jh < _HARD_K)) |
                 ((jh >= lab16 + _M) & (jh < _HARD_K + _M)))
        hard_ref[...] = jnp.where(hmask, 1.0, 0.0)

    c = cen_ref[...]
    s = jax.lax.dot_general(x_ref[...], c, (((1,), (1,)), ((), ())),
                            preferred_element_type=jnp.float32)
    e = jnp.exp(s * (1.0 / _T))

    ecam = e * camm_ref[...]
    pos_mask = jdiv_ref[...] == (lab - i * (_BLK // _M))
    zero = jnp.zeros_like(e)
    fd = jnp.sum(ecam, axis=1, keepdims=True)
    pd = jnp.sum(jnp.where(pos_mask, e, zero), axis=1, keepdims=True)
    up = jnp.sum(jnp.where(pos_mask, ecam, zero), axis=1, keepdims=True)
    hscale = jnp.where(i == 0, 1.0, 0.0)
    hd = jnp.sum(e[:, :128] * hard_ref[...], axis=1, keepdims=True) * hscale
    acc_ref[...] += jnp.concatenate([fd, pd, up, hd], axis=1)

    @pl.when(i == _NB - 1)
    def _finish():
        acc = acc_ref[...]
        log_up = jnp.log(acc[:, 2:3])
        log_fd = jnp.log(acc[:, 0:1])
        log_pd = jnp.log(acc[:, 1:2] + acc[:, 3:4])
        intra = -jnp.sum(log_up - log_fd)
        inter = -jnp.sum(log_up - log_pd)
        gate = (gate_ref[...] >= 5).astype(jnp.float32)
        out_ref[...] = jnp.concatenate(
            [intra.reshape(1, 1), (_LAMDA * inter).reshape(1, 1)],
            axis=1) * gate


def kernel(feats, centers, labels, camids, epoch):
    lab = labels.reshape(_B, 1).astype(jnp.int32)
    cam = camids.reshape(_B, 1).astype(jnp.int32)
    ep = jnp.asarray(epoch, jnp.int32).reshape(1, 1)
    out = pl.pallas_call(
        _loss_kernel,
        grid=(_NB,),
        in_specs=[
            pl.BlockSpec((_B, _D), lambda i: (0, 0)),
            pl.BlockSpec((_B, 1), lambda i: (0, 0)),
            pl.BlockSpec((_B, 1), lambda i: (0, 0)),
            pl.BlockSpec((1, 1), lambda i: (0, 0)),
            pl.BlockSpec((_BLK, _D), lambda i: (i, 0)),
        ],
        out_specs=pl.BlockSpec((1, 2), lambda i: (0, 0)),
        out_shape=jax.ShapeDtypeStruct((1, 2), jnp.float32),
        scratch_shapes=[
            pltpu.VMEM((_B, _D), jnp.float32),
            pltpu.VMEM((_B, _BLK), jnp.float32),
            pltpu.VMEM((_B, _BLK), jnp.int32),
            pltpu.VMEM((_B, 4), jnp.float32),
            pltpu.VMEM((_B, 128), jnp.float32),
        ],
        compiler_params=pltpu.CompilerParams(
            dimension_semantics=("arbitrary",)),
    )(feats, lab, cam, ep, centers)
    return out.reshape(2)
